# Initial kernel scaffold; baseline (speedup 1.0000x reference)
import jax
import jax.numpy as jnp
from jax.experimental import pallas as pl


def kernel(input_ids, token_type_ids, word_embeddings, position_embeddings, token_type_embeddings, gamma, beta):
    raise NotImplementedError("write your pallas kernel here")


# speedup vs baseline: 1.1255x; 1.1255x over previous
import functools

import jax
import jax.numpy as jnp
from jax import lax
from jax.experimental import pallas as pl
from jax.experimental.pallas import tpu as pltpu
from jax.experimental.pallas import tpu_sc as plsc

LANES = 16
CHUNK = 64
EPS = 1e-12


def _lane_sum(x):
    iota = lax.iota(jnp.int32, LANES)
    dnums = lax.GatherDimensionNumbers(
        offset_dims=(), collapsed_slice_dims=(0,), start_index_map=(0,))
    for k in (8, 4, 2, 1):
        perm = lax.bitwise_xor(iota, k).reshape(LANES, 1)
        x = x + lax.gather(x, perm, dnums, slice_sizes=(1,),
                           mode=lax.GatherScatterMode.PROMISE_IN_BOUNDS)
    return x


@functools.lru_cache(maxsize=None)
def _make_sc_kernel(n_tokens, seq, hidden, n_comb_rows):
    info = plsc.get_sparse_core_info()
    n_workers = info.num_cores * info.num_subcores
    assert n_tokens % (n_workers * seq) == 0, "each worker owns whole sequences"
    tok_per_w = n_tokens // n_workers
    n_chunks = tok_per_w // CHUNK
    assert seq % CHUNK == 0 and hidden % LANES == 0
    hchunks = hidden // LANES
    chunks_per_seq = seq // CHUNK

    @functools.partial(
        pl.kernel,
        out_type=jax.ShapeDtypeStruct((n_tokens, hidden), jnp.float32),
        mesh=plsc.VectorSubcoreMesh(core_axis_name="c", subcore_axis_name="s"),
        scratch_types=[
            pltpu.VMEM((CHUNK,), jnp.int32),
            pltpu.VMEM((CHUNK,), jnp.int32),
            pltpu.VMEM((CHUNK, hidden), jnp.float32),
            pltpu.VMEM((CHUNK, hidden), jnp.float32),
            pltpu.VMEM((hidden,), jnp.float32),
            pltpu.VMEM((hidden,), jnp.float32),
            pltpu.SemaphoreType.DMA,
            pltpu.SemaphoreType.DMA,
        ],
    )
    def sc_kernel(ids_hbm, tids_hbm, word_hbm, comb_hbm, gamma_hbm, beta_hbm,
                  out_hbm, idx_v, cidx_v, rows_v, comb_v, gamma_v, beta_v,
                  sem_w, sem_c):
        wid = lax.axis_index("s") * info.num_cores + lax.axis_index("c")
        w_base = wid * tok_per_w
        pltpu.sync_copy(gamma_hbm, gamma_v)
        pltpu.sync_copy(beta_hbm, beta_v)

        inv_h = jnp.float32(1.0 / hidden)

        def chunk_body(k, carry):
            base = w_base + k * CHUNK
            p0 = lax.rem(k, chunks_per_seq) * CHUNK
            pltpu.sync_copy(ids_hbm.at[pl.ds(base, CHUNK)], idx_v)
            pltpu.sync_copy(tids_hbm.at[pl.ds(base, CHUNK)], cidx_v)
            for j in range(CHUNK // LANES):
                sl = pl.ds(j * LANES, LANES)
                loc = lax.iota(jnp.int32, LANES) + (p0 + j * LANES)
                cidx_v[sl] = cidx_v[sl] * seq + loc
            cp_w = pltpu.async_copy(word_hbm.at[idx_v], rows_v, sem_w)
            cp_c = pltpu.async_copy(comb_hbm.at[cidx_v], comb_v, sem_c)
            cp_w.wait()
            cp_c.wait()

            def row_body(r, rcarry):
                acc = jnp.zeros((LANES,), jnp.float32)
                accsq = jnp.zeros((LANES,), jnp.float32)
                for c in range(hchunks):
                    sl = pl.ds(c * LANES, LANES)
                    x = rows_v[r, sl] + comb_v[r, sl]
                    rows_v[r, sl] = x
                    acc = acc + x
                    accsq = accsq + x * x
                mean_v = _lane_sum(acc) * inv_h
                var_v = _lane_sum(accsq) * inv_h - mean_v * mean_v
                v = var_v + EPS
                bits = lax.bitcast_convert_type(v, jnp.int32)
                ones = jnp.full((LANES,), 1, jnp.int32)
                bits = 0x5F3759DF - lax.shift_right_logical(bits, ones)
                y = lax.bitcast_convert_type(bits, jnp.float32)
                half = v * 0.5
                for _ in range(3):
                    y = y * (1.5 - half * y * y)
                for c in range(hchunks):
                    sl = pl.ds(c * LANES, LANES)
                    xhat = (rows_v[r, sl] - mean_v) * y
                    rows_v[r, sl] = xhat * gamma_v[sl] + beta_v[sl]
                return rcarry

            lax.fori_loop(0, CHUNK, row_body, 0)
            pltpu.sync_copy(rows_v, out_hbm.at[pl.ds(base, CHUNK)])
            return carry

        lax.fori_loop(0, n_chunks, chunk_body, 0)

    return sc_kernel


def kernel(input_ids, token_type_ids, word_embeddings, position_embeddings,
           token_type_embeddings, gamma, beta):
    batch, seq = input_ids.shape
    hidden = word_embeddings.shape[1]
    n_tokens = batch * seq
    ids = input_ids.reshape(-1).astype(jnp.int32)
    tids = token_type_ids.reshape(-1).astype(jnp.int32)
    comb = (token_type_embeddings[:, None, :]
            + position_embeddings[None, :seq, :]).reshape(-1, hidden)
    sc = _make_sc_kernel(n_tokens, seq, hidden, comb.shape[0])
    out = sc(ids, tids, word_embeddings, comb, gamma, beta)
    return out.reshape(batch, seq, hidden)
